# baseline (speedup 1.0000x reference)
import jax
import jax.numpy as jnp
from jax import lax
from jax.experimental import pallas as pl
from jax.experimental.pallas import tpu as pltpu
from jax.experimental.pallas import tpu_sc as plsc

VOCAB = 1000000
DIM = 64
MAXLEN = 200
BATCH = 4096

NC, NS, L = 2, 16, 16
NW = NC * NS
CB = 128
NLB = CB // L


def _body(xT_hbm, tok2_hbm, pos_hbm, out_hbm,
          idxall, pidx0, pidx1, gb0, gb1, g2b0, g2b1, pos_v,
          sg0, sg1, ss0, ss1):
    wid = lax.axis_index("s") * NC + lax.axis_index("c")
    bbase = wid * CB
    pidxs, gbs, g2bs = (pidx0, pidx1), (gb0, gb1), (g2b0, g2b1)
    sgs, sss = (sg0, sg1), (ss0, ss1)

    pltpu.sync_copy(xT_hbm.at[:, pl.ds(bbase, CB)], idxall)
    pltpu.sync_copy(pos_hbm, pos_v)

    iota = lax.iota(jnp.int32, L)

    def compute_pidx(t, slot):
        for j in range(NLB):
            s = pl.ds(j * L, L)
            pidxs[slot][s] = lax.shift_right_logical(idxall[t, s], 1)

    def gather_start(slot):
        pltpu.async_copy(tok2_hbm.at[pidxs[slot]], gbs[slot], sgs[slot])

    def gather_wait(slot):
        pltpu.make_async_copy(tok2_hbm.at[pidxs[slot]], gbs[slot],
                              sgs[slot]).wait()

    def store_start(t, slot):
        pltpu.async_copy(g2bs[slot], out_hbm.at[t, :, pl.ds(bbase, CB)],
                         sss[slot])

    def store_wait(slot):
        pltpu.make_async_copy(g2bs[slot], out_hbm.at[0, :, pl.ds(bbase, CB)],
                              sss[slot]).wait()

    compute_pidx(0, 0)
    gather_start(0)

    @pl.loop(0, MAXLEN, step=2)
    def _pair(k):
        for b in range(2):
            cur = k + b
            nb = 1 - b

            @pl.when(cur + 1 < MAXLEN)
            def _():
                compute_pidx(cur + 1, nb)

                @pl.when(cur >= 1)
                def _():
                    store_wait(nb)
                gather_start(nb)

            gather_wait(b)

            gb, g2b = gbs[b], g2bs[b]
            rows, cbases = [], []
            for j in range(NLB):
                v = idxall[cur, pl.ds(j * L, L)]
                rows.append(iota + j * L)
                cbases.append(lax.shift_left(lax.bitwise_and(v, 1), 6))

            zero = iota * 0
            curbase = zero + cur * DIM

            @pl.loop(0, DIM, unroll=8)
            def _d(d):
                ps = plsc.load_gather(pos_v, [curbase + d])
                for j in range(NLB):
                    val = plsc.load_gather(gb, [rows[j], cbases[j] + d])
                    g2b[d, pl.ds(j * L, L)] = val + ps

            store_start(cur, b)

    store_wait(0)
    store_wait(1)


@jax.jit
def _run(xT, tok2, pos_table):
    mesh = plsc.VectorSubcoreMesh(core_axis_name="c", subcore_axis_name="s")
    return pl.kernel(
        _body,
        out_type=jax.ShapeDtypeStruct((MAXLEN, DIM, BATCH), jnp.float32),
        mesh=mesh,
        scratch_types=[
            pltpu.VMEM((MAXLEN, CB), jnp.int32),
            pltpu.VMEM((CB,), jnp.int32),
            pltpu.VMEM((CB,), jnp.int32),
            pltpu.VMEM((CB, 2 * DIM), jnp.float32),
            pltpu.VMEM((CB, 2 * DIM), jnp.float32),
            pltpu.VMEM((DIM, CB), jnp.float32),
            pltpu.VMEM((DIM, CB), jnp.float32),
            pltpu.VMEM((MAXLEN * DIM,), jnp.float32),
            pltpu.SemaphoreType.DMA,
            pltpu.SemaphoreType.DMA,
            pltpu.SemaphoreType.DMA,
            pltpu.SemaphoreType.DMA,
        ],
        compiler_params=pltpu.CompilerParams(use_tc_tiling_on_sc=True,
                                             needs_layout_passes=False),
    )(xT, tok2, pos_table)


def kernel(x, token_table, pos_table):
    xT = jnp.swapaxes(x.astype(jnp.int32), 0, 1)
    tok2 = token_table.reshape(VOCAB // 2, 2 * DIM)
    outP = _run(xT, tok2, pos_table.reshape(-1))
    return jnp.transpose(outP, (2, 0, 1))


# speedup vs baseline: 1.4868x; 1.4868x over previous
import jax
import jax.numpy as jnp
from jax import lax
from jax.experimental import pallas as pl
from jax.experimental.pallas import tpu as pltpu
from jax.experimental.pallas import tpu_sc as plsc

VOCAB = 1000000
DIM = 64
MAXLEN = 200
BATCH = 4096

NC, NS, L = 2, 16, 16
NW = NC * NS
CB = 128
NLB = CB // L


def _body(xT_hbm, tok2_hbm, pos_hbm, out_hbm,
          idxall, pidx0, pidx1, gb0, gb1, g2b0, g2b1, pos_v,
          sg0, sg1, ss0, ss1):
    wid = lax.axis_index("s") * NC + lax.axis_index("c")
    bbase = wid * CB
    pidxs, gbs, g2bs = (pidx0, pidx1), (gb0, gb1), (g2b0, g2b1)
    sgs, sss = (sg0, sg1), (ss0, ss1)

    pltpu.sync_copy(xT_hbm.at[:, pl.ds(bbase, CB)], idxall)
    pltpu.sync_copy(pos_hbm, pos_v)

    iota = lax.iota(jnp.int32, L)

    def compute_pidx(t, slot):
        for j in range(NLB):
            s = pl.ds(j * L, L)
            pidxs[slot][s] = lax.shift_right_logical(idxall[t, s], 1)

    def gather_start(slot):
        pltpu.async_copy(tok2_hbm.at[pidxs[slot]], gbs[slot], sgs[slot])

    def gather_wait(slot):
        pltpu.make_async_copy(tok2_hbm.at[pidxs[slot]], gbs[slot],
                              sgs[slot]).wait()

    def store_start(t, slot):
        pltpu.async_copy(g2bs[slot], out_hbm.at[t, :, pl.ds(bbase, CB)],
                         sss[slot])

    def store_wait(slot):
        pltpu.make_async_copy(g2bs[slot], out_hbm.at[0, :, pl.ds(bbase, CB)],
                              sss[slot]).wait()

    compute_pidx(0, 0)
    gather_start(0)

    @pl.loop(0, MAXLEN, step=2)
    def _pair(k):
        for b in range(2):
            cur = k + b
            nb = 1 - b

            @pl.when(cur + 1 < MAXLEN)
            def _():
                compute_pidx(cur + 1, nb)

                @pl.when(cur >= 1)
                def _():
                    store_wait(nb)
                gather_start(nb)

            gather_wait(b)

            gb, g2b = gbs[b], g2bs[b]
            rows, cbases = [], []
            for j in range(NLB):
                v = idxall[cur, pl.ds(j * L, L)]
                rows.append(iota + j * L)
                cbases.append(lax.shift_left(lax.bitwise_and(v, 1), 6))

            zero = iota * 0
            curbase = zero + cur * DIM

            @plsc.parallel_loop(0, DIM, unroll=8)
            def _d(d):
                ps = plsc.load_gather(pos_v, [curbase + d])
                for j in range(NLB):
                    val = plsc.load_gather(gb, [rows[j], cbases[j] + d])
                    g2b[d, pl.ds(j * L, L)] = val + ps

            store_start(cur, b)

    store_wait(0)
    store_wait(1)


@jax.jit
def _run(xT, tok2, pos_table):
    mesh = plsc.VectorSubcoreMesh(core_axis_name="c", subcore_axis_name="s")
    return pl.kernel(
        _body,
        out_type=jax.ShapeDtypeStruct((MAXLEN, DIM, BATCH), jnp.float32),
        mesh=mesh,
        scratch_types=[
            pltpu.VMEM((MAXLEN, CB), jnp.int32),
            pltpu.VMEM((CB,), jnp.int32),
            pltpu.VMEM((CB,), jnp.int32),
            pltpu.VMEM((CB, 2 * DIM), jnp.float32),
            pltpu.VMEM((CB, 2 * DIM), jnp.float32),
            pltpu.VMEM((DIM, CB), jnp.float32),
            pltpu.VMEM((DIM, CB), jnp.float32),
            pltpu.VMEM((MAXLEN * DIM,), jnp.float32),
            pltpu.SemaphoreType.DMA,
            pltpu.SemaphoreType.DMA,
            pltpu.SemaphoreType.DMA,
            pltpu.SemaphoreType.DMA,
        ],
        compiler_params=pltpu.CompilerParams(use_tc_tiling_on_sc=True,
                                             needs_layout_passes=False),
    )(xT, tok2, pos_table)


def kernel(x, token_table, pos_table):
    xT = jnp.swapaxes(x.astype(jnp.int32), 0, 1)
    tok2 = token_table.reshape(VOCAB // 2, 2 * DIM)
    outP = _run(xT, tok2, pos_table.reshape(-1))
    return jnp.transpose(outP, (2, 0, 1))
